# baseline (speedup 1.0000x reference)
import functools

import jax
import jax.numpy as jnp
import numpy as np
from jax import lax
from jax.experimental import pallas as pl
from jax.experimental.pallas import tpu as pltpu
from jax.experimental.pallas import tpu_sc as plsc

_B = 64
_C = 3
_H = 128
_W = 128
_Q = 512
_K = 21

_D = _C * _H * _W
_KD = 441
_KDP = 512

_NW = 32
_KA_W = (_Q - _B) // 16

_IDX = np.array([
    121, 480, 35, 130, 263, 148, 197, 410, 398, 45, 176, 462, 446, 366, 257,
    179, 139, 315, 501, 188, 312, 499, 318, 448, 304, 99, 309, 144, 152, 189,
    487, 325, 31, 112, 495, 356, 493, 507, 268, 429, 409, 85, 63, 117, 417,
    174, 441, 509, 481, 272, 114, 254, 82, 65, 7, 350, 4, 101, 463, 452, 444,
    102, 78, 163, 157, 302, 183, 29, 240, 177, 278, 259, 108, 305, 83, 129,
    367, 212, 277, 504, 300, 44, 211, 16, 58, 123, 37, 336, 111, 19, 61, 447,
    2, 142, 34, 369, 339, 156, 436, 5, 461, 415, 90, 363, 175, 167, 284, 379,
    251, 110, 72, 155, 178, 323, 291, 388, 269, 354, 368, 219, 510, 153, 30,
    275, 42, 186, 342, 406, 468, 439, 307, 256, 419, 246, 3, 362, 380, 327,
    393, 70, 378, 400, 271, 488, 311, 67, 273, 223, 422, 39, 56, 274, 192,
    169, 349, 218, 195, 476, 173, 245, 241, 69, 383, 80, 22, 6, 321, 199, 345,
    118, 235, 54, 442, 479, 423, 266, 77, 425, 147, 18, 340, 298, 249, 294,
    375, 382, 10, 11, 234, 53, 236, 455, 94, 332, 511, 331, 437, 353, 489,
    287, 32, 217, 283, 355, 407, 159, 440, 15, 470, 184, 49, 137, 50, 138, 20,
    445, 237, 280, 253, 185, 460, 43, 389, 335, 258, 370, 344, 92, 8, 503,
    324, 140, 233, 24, 81, 239, 314, 453, 96, 475, 467, 154, 135, 472, 490,
    469, 500, 264, 160, 106, 128, 265, 426, 386, 191, 9, 200, 40, 187, 71,
    346, 438, 333, 248, 164, 207, 93, 59, 201, 158, 210, 420, 402, 75, 508,
    131, 411, 97, 66, 25, 196, 424, 364, 497, 242, 338, 206, 243, 397, 341,
    450, 414, 238, 295, 432, 431, 308, 73, 320, 13, 52, 491, 203, 289, 303,
    202, 255, 194, 88, 250, 337, 62, 230, 150, 261, 330, 262, 209, 132, 357,
    87, 76, 198, 486, 60, 244, 457, 47, 392, 374, 276, 33, 79, 451, 180, 403,
    247, 14, 459, 286, 421, 458, 228, 17, 38, 86, 231, 190, 232, 482, 23, 105,
    484, 395, 427, 301, 474, 376, 405, 494, 471, 391, 313, 220, 0, 473, 145,
    371, 213, 226, 381, 133, 281, 41, 64, 416, 21, 443, 161, 279, 285, 166,
    124, 116, 449, 26, 165, 168, 193, 57, 208, 181, 89, 146, 182, 126, 125,
    297, 1, 115, 28, 113, 225, 361, 351, 465, 172, 377, 162, 48, 170, 466,
    505, 227, 36, 252, 502, 492, 119, 151, 385, 306, 120, 372, 390, 224, 122,
    270, 100, 418, 433, 329, 365, 396, 91, 222, 55, 496, 498, 103, 51, 293,
    215, 384, 127, 98, 483, 506, 282, 107, 27, 322, 74, 136, 229, 319, 328,
    430, 343, 204, 221, 296, 12, 134, 454, 477, 408, 109, 84, 428, 317, 358,
    394, 299, 205, 171, 288, 143, 68, 267, 216, 435, 149, 485, 434, 141, 464,
    334, 404, 104, 352, 95, 387, 316, 214, 290, 46, 310, 348, 401, 260, 478,
    292, 359, 326, 347, 456, 399, 373, 412, 360, 413], dtype=np.int64)

_KIDX = np.concatenate([_IDX[_B:], _IDX[:_B]]).astype(np.int32)

_TASKS = []
for _j in range(_B):
    _TASKS.append((int(_IDX[_j]), 1, _j))
for _j in range(_B, _Q):
    _TASKS.append((int(_IDX[_j]), 0, _j))
_PER_W = len(_TASKS) // _NW

_mesh = plsc.VectorSubcoreMesh(core_axis_name="c", subcore_axis_name="s")

_img_out = [
    jax.ShapeDtypeStruct((_Q, _D), jnp.float32),
    jax.ShapeDtypeStruct((_B, _D), jnp.float32),
]


@functools.partial(pl.kernel, out_type=_img_out, mesh=_mesh,
                   scratch_types=[pltpu.SemaphoreType.DMA,
                                  pltpu.SemaphoreType.DMA])
def _sc_img_stream(tbl, batch, newt, deqt, gsem, hsem):
    w = lax.axis_index("s") * 2 + lax.axis_index("c")

    h0 = pltpu.async_copy(batch.at[pl.ds(w * 2, 2)],
                          newt.at[pl.ds(w * 2, 2)], hsem)

    dsts = {0: newt, 1: deqt}

    def issue(wi):
        for (src, dst_id, dj) in _TASKS[wi::_NW]:
            pltpu.async_copy(tbl.at[pl.ds(src, 1)],
                             dsts[dst_id].at[pl.ds(dj, 1)], gsem)

    for wi in range(_NW):
        @pl.when(w == wi)
        def _(wi=wi):
            issue(wi)

    pltpu.make_async_copy(tbl.at[pl.ds(0, _PER_W)],
                          newt.at[pl.ds(0, _PER_W)], gsem).wait()
    h0.wait()


@functools.partial(
    pl.kernel,
    out_type=[
        jax.ShapeDtypeStruct((_Q, _KDP), jnp.float32),
        jax.ShapeDtypeStruct((_B, _KDP), jnp.float32),
    ],
    mesh=_mesh,
    scratch_types=[
        pltpu.VMEM((16,), jnp.int32),
        pltpu.VMEM((16, _KDP), jnp.float32),
        pltpu.SemaphoreType.DMA,
    ],
)
def _sc_ker_stream(kidx, ker2, lr2, newker2, deqker2, kidx_v, kbuf, hsem):
    w = lax.axis_index("s") * 2 + lax.axis_index("c")

    h0 = pltpu.async_copy(lr2.at[pl.ds(w * 2, 2)],
                          newker2.at[pl.ds(w * 2, 2)], hsem)

    def ker_chunk(idx_off, dst, dst_off):
        pltpu.sync_copy(kidx.at[pl.ds(idx_off, 16)], kidx_v)
        pltpu.async_copy(ker2.at[kidx_v], kbuf, hsem).wait()
        pltpu.sync_copy(kbuf, dst.at[pl.ds(dst_off, 16)])

    @pl.when(w < _KA_W)
    def _():
        ker_chunk(w * 16, newker2, _B + w * 16)

    @pl.when(w >= _KA_W)
    def _():
        ker_chunk((_Q - _B) + (w - _KA_W) * 16, deqker2, (w - _KA_W) * 16)

    h0.wait()


def kernel(query, key_img, lr_gt_kernel, queue_q, queue_k, queue_ker):
    q3 = queue_q.reshape(_Q, _D)
    k3 = queue_k.reshape(_Q, _D)
    ker2 = jnp.pad(queue_ker.reshape(_Q, _KD), ((0, 0), (0, _KDP - _KD)))
    query3 = query.reshape(_B, _D)
    keyimg3 = key_img.reshape(_B, _D)
    lr2 = jnp.pad(lr_gt_kernel.reshape(_B, _KD), ((0, 0), (0, _KDP - _KD)))
    kidx = jnp.asarray(_KIDX)

    newq3, deqq3 = _sc_img_stream(q3, query3)
    newk3, deqk3 = _sc_img_stream(k3, keyimg3)
    newker2, deqker2 = _sc_ker_stream(kidx, ker2, lr2)

    new_qq = newq3.reshape(_Q, _C, _H, _W)
    new_qk = newk3.reshape(_Q, _C, _H, _W)
    new_qker = newker2[:, :_KD].reshape(_Q, 1, _K, _K)
    q_deq = deqq3.reshape(_B, _C, _H, _W)
    k_deq = deqk3.reshape(_B, _C, _H, _W)
    ker_deq = deqker2[:, :_KD].reshape(_B, 1, _K, _K)
    return (q_deq, k_deq, ker_deq, new_qq, new_qk, new_qker)


# speedup vs baseline: 12.9660x; 12.9660x over previous
import functools

import jax
import jax.numpy as jnp
import numpy as np
from jax import lax
from jax.experimental import pallas as pl
from jax.experimental.pallas import tpu as pltpu
from jax.experimental.pallas import tpu_sc as plsc

_B = 64
_C = 3
_H = 128
_W = 128
_Q = 512
_K = 21

_D = _C * _H * _W
_KD = 441
_KDP = 512

_NW = 32
_KA_W = (_Q - _B) // 16

_IDX = np.array([
    121, 480, 35, 130, 263, 148, 197, 410, 398, 45, 176, 462, 446, 366, 257,
    179, 139, 315, 501, 188, 312, 499, 318, 448, 304, 99, 309, 144, 152, 189,
    487, 325, 31, 112, 495, 356, 493, 507, 268, 429, 409, 85, 63, 117, 417,
    174, 441, 509, 481, 272, 114, 254, 82, 65, 7, 350, 4, 101, 463, 452, 444,
    102, 78, 163, 157, 302, 183, 29, 240, 177, 278, 259, 108, 305, 83, 129,
    367, 212, 277, 504, 300, 44, 211, 16, 58, 123, 37, 336, 111, 19, 61, 447,
    2, 142, 34, 369, 339, 156, 436, 5, 461, 415, 90, 363, 175, 167, 284, 379,
    251, 110, 72, 155, 178, 323, 291, 388, 269, 354, 368, 219, 510, 153, 30,
    275, 42, 186, 342, 406, 468, 439, 307, 256, 419, 246, 3, 362, 380, 327,
    393, 70, 378, 400, 271, 488, 311, 67, 273, 223, 422, 39, 56, 274, 192,
    169, 349, 218, 195, 476, 173, 245, 241, 69, 383, 80, 22, 6, 321, 199, 345,
    118, 235, 54, 442, 479, 423, 266, 77, 425, 147, 18, 340, 298, 249, 294,
    375, 382, 10, 11, 234, 53, 236, 455, 94, 332, 511, 331, 437, 353, 489,
    287, 32, 217, 283, 355, 407, 159, 440, 15, 470, 184, 49, 137, 50, 138, 20,
    445, 237, 280, 253, 185, 460, 43, 389, 335, 258, 370, 344, 92, 8, 503,
    324, 140, 233, 24, 81, 239, 314, 453, 96, 475, 467, 154, 135, 472, 490,
    469, 500, 264, 160, 106, 128, 265, 426, 386, 191, 9, 200, 40, 187, 71,
    346, 438, 333, 248, 164, 207, 93, 59, 201, 158, 210, 420, 402, 75, 508,
    131, 411, 97, 66, 25, 196, 424, 364, 497, 242, 338, 206, 243, 397, 341,
    450, 414, 238, 295, 432, 431, 308, 73, 320, 13, 52, 491, 203, 289, 303,
    202, 255, 194, 88, 250, 337, 62, 230, 150, 261, 330, 262, 209, 132, 357,
    87, 76, 198, 486, 60, 244, 457, 47, 392, 374, 276, 33, 79, 451, 180, 403,
    247, 14, 459, 286, 421, 458, 228, 17, 38, 86, 231, 190, 232, 482, 23, 105,
    484, 395, 427, 301, 474, 376, 405, 494, 471, 391, 313, 220, 0, 473, 145,
    371, 213, 226, 381, 133, 281, 41, 64, 416, 21, 443, 161, 279, 285, 166,
    124, 116, 449, 26, 165, 168, 193, 57, 208, 181, 89, 146, 182, 126, 125,
    297, 1, 115, 28, 113, 225, 361, 351, 465, 172, 377, 162, 48, 170, 466,
    505, 227, 36, 252, 502, 492, 119, 151, 385, 306, 120, 372, 390, 224, 122,
    270, 100, 418, 433, 329, 365, 396, 91, 222, 55, 496, 498, 103, 51, 293,
    215, 384, 127, 98, 483, 506, 282, 107, 27, 322, 74, 136, 229, 319, 328,
    430, 343, 204, 221, 296, 12, 134, 454, 477, 408, 109, 84, 428, 317, 358,
    394, 299, 205, 171, 288, 143, 68, 267, 216, 435, 149, 485, 434, 141, 464,
    334, 404, 104, 352, 95, 387, 316, 214, 290, 46, 310, 348, 401, 260, 478,
    292, 359, 326, 347, 456, 399, 373, 412, 360, 413], dtype=np.int64)

_KIDX = np.concatenate([_IDX[_B:], _IDX[:_B]]).astype(np.int32)

_INV = np.argsort(_IDX)

_mesh = plsc.VectorSubcoreMesh(core_axis_name="c", subcore_axis_name="s")

_img_out = [
    jax.ShapeDtypeStruct((_Q, _D), jnp.float32),
    jax.ShapeDtypeStruct((_B, _D), jnp.float32),
]

_ROWS_W = _Q // _NW
_HEAD_W = _B // _NW


@functools.partial(pl.kernel, out_type=_img_out, mesh=_mesh,
                   scratch_types=[
                       pltpu.VMEM_SHARED((16, 2, 1, _D), jnp.float32),
                       pltpu.SemaphoreType.DMA,
                       pltpu.SemaphoreType.DMA])
def _sc_img_stream(tbl, batch, newt, deqt, slots, lsem, wsem):
    w = lax.axis_index("s") * 2 + lax.axis_index("c")

    def worker_prog(wi):
        sid = wi // 2
        tasks = []
        for s in range(wi * _ROWS_W, (wi + 1) * _ROWS_W):
            j = int(_INV[s])
            if j < _B:
                tasks.append((tbl, s, deqt, j))
            else:
                tasks.append((tbl, s, newt, j))
        for r in range(wi * _HEAD_W, (wi + 1) * _HEAD_W):
            tasks.append((batch, r, newt, r))

        n = len(tasks)
        hl = [None] * n
        hw = [None] * n
        for i in range(n + 1):
            if i < n:
                if i >= 2:
                    hw[i - 2].wait()
                src_ref, s, _, _ = tasks[i]
                hl[i] = pltpu.async_copy(src_ref.at[pl.ds(s, 1)],
                                         slots.at[sid, i % 2], lsem)
            if i >= 1:
                _, _, dst_ref, j = tasks[i - 1]
                hl[i - 1].wait()
                hw[i - 1] = pltpu.async_copy(slots.at[sid, (i - 1) % 2],
                                             dst_ref.at[pl.ds(j, 1)], wsem)
        hw[n - 2].wait()
        hw[n - 1].wait()

    for wi in range(_NW):
        @pl.when(w == wi)
        def _(wi=wi):
            worker_prog(wi)


@functools.partial(
    pl.kernel,
    out_type=[
        jax.ShapeDtypeStruct((_Q, _KDP), jnp.float32),
        jax.ShapeDtypeStruct((_B, _KDP), jnp.float32),
    ],
    mesh=_mesh,
    scratch_types=[
        pltpu.VMEM((16,), jnp.int32),
        pltpu.VMEM((16, _KDP), jnp.float32),
        pltpu.SemaphoreType.DMA,
    ],
)
def _sc_ker_stream(kidx, ker2, lr2, newker2, deqker2, kidx_v, kbuf, hsem):
    w = lax.axis_index("s") * 2 + lax.axis_index("c")

    h0 = pltpu.async_copy(lr2.at[pl.ds(w * 2, 2)],
                          newker2.at[pl.ds(w * 2, 2)], hsem)

    def ker_chunk(idx_off, dst, dst_off):
        pltpu.sync_copy(kidx.at[pl.ds(idx_off, 16)], kidx_v)
        pltpu.async_copy(ker2.at[kidx_v], kbuf, hsem).wait()
        pltpu.sync_copy(kbuf, dst.at[pl.ds(dst_off, 16)])

    @pl.when(w < _KA_W)
    def _():
        ker_chunk(w * 16, newker2, _B + w * 16)

    @pl.when(w >= _KA_W)
    def _():
        ker_chunk((_Q - _B) + (w - _KA_W) * 16, deqker2, (w - _KA_W) * 16)

    h0.wait()


def kernel(query, key_img, lr_gt_kernel, queue_q, queue_k, queue_ker):
    q3 = queue_q.reshape(_Q, _D)
    k3 = queue_k.reshape(_Q, _D)
    ker2 = jnp.pad(queue_ker.reshape(_Q, _KD), ((0, 0), (0, _KDP - _KD)))
    query3 = query.reshape(_B, _D)
    keyimg3 = key_img.reshape(_B, _D)
    lr2 = jnp.pad(lr_gt_kernel.reshape(_B, _KD), ((0, 0), (0, _KDP - _KD)))
    kidx = jnp.asarray(_KIDX)

    newq3, deqq3 = _sc_img_stream(q3, query3)
    newk3, deqk3 = _sc_img_stream(k3, keyimg3)
    newker2, deqker2 = _sc_ker_stream(kidx, ker2, lr2)

    new_qq = newq3.reshape(_Q, _C, _H, _W)
    new_qk = newk3.reshape(_Q, _C, _H, _W)
    new_qker = newker2[:, :_KD].reshape(_Q, 1, _K, _K)
    q_deq = deqq3.reshape(_B, _C, _H, _W)
    k_deq = deqk3.reshape(_B, _C, _H, _W)
    ker_deq = deqker2[:, :_KD].reshape(_B, 1, _K, _K)
    return (q_deq, k_deq, ker_deq, new_qq, new_qk, new_qker)


# speedup vs baseline: 35.0357x; 2.7021x over previous
import functools

import jax
import jax.numpy as jnp
import numpy as np
from jax import lax
from jax.experimental import pallas as pl
from jax.experimental.pallas import tpu as pltpu
from jax.experimental.pallas import tpu_sc as plsc

_B = 64
_C = 3
_H = 128
_W = 128
_Q = 512
_K = 21

_D = _C * _H * _W
_KD = 441
_KDP = 512

_NW = 32
_KA_W = (_Q - _B) // 16

_IDX = np.array([
    121, 480, 35, 130, 263, 148, 197, 410, 398, 45, 176, 462, 446, 366, 257,
    179, 139, 315, 501, 188, 312, 499, 318, 448, 304, 99, 309, 144, 152, 189,
    487, 325, 31, 112, 495, 356, 493, 507, 268, 429, 409, 85, 63, 117, 417,
    174, 441, 509, 481, 272, 114, 254, 82, 65, 7, 350, 4, 101, 463, 452, 444,
    102, 78, 163, 157, 302, 183, 29, 240, 177, 278, 259, 108, 305, 83, 129,
    367, 212, 277, 504, 300, 44, 211, 16, 58, 123, 37, 336, 111, 19, 61, 447,
    2, 142, 34, 369, 339, 156, 436, 5, 461, 415, 90, 363, 175, 167, 284, 379,
    251, 110, 72, 155, 178, 323, 291, 388, 269, 354, 368, 219, 510, 153, 30,
    275, 42, 186, 342, 406, 468, 439, 307, 256, 419, 246, 3, 362, 380, 327,
    393, 70, 378, 400, 271, 488, 311, 67, 273, 223, 422, 39, 56, 274, 192,
    169, 349, 218, 195, 476, 173, 245, 241, 69, 383, 80, 22, 6, 321, 199, 345,
    118, 235, 54, 442, 479, 423, 266, 77, 425, 147, 18, 340, 298, 249, 294,
    375, 382, 10, 11, 234, 53, 236, 455, 94, 332, 511, 331, 437, 353, 489,
    287, 32, 217, 283, 355, 407, 159, 440, 15, 470, 184, 49, 137, 50, 138, 20,
    445, 237, 280, 253, 185, 460, 43, 389, 335, 258, 370, 344, 92, 8, 503,
    324, 140, 233, 24, 81, 239, 314, 453, 96, 475, 467, 154, 135, 472, 490,
    469, 500, 264, 160, 106, 128, 265, 426, 386, 191, 9, 200, 40, 187, 71,
    346, 438, 333, 248, 164, 207, 93, 59, 201, 158, 210, 420, 402, 75, 508,
    131, 411, 97, 66, 25, 196, 424, 364, 497, 242, 338, 206, 243, 397, 341,
    450, 414, 238, 295, 432, 431, 308, 73, 320, 13, 52, 491, 203, 289, 303,
    202, 255, 194, 88, 250, 337, 62, 230, 150, 261, 330, 262, 209, 132, 357,
    87, 76, 198, 486, 60, 244, 457, 47, 392, 374, 276, 33, 79, 451, 180, 403,
    247, 14, 459, 286, 421, 458, 228, 17, 38, 86, 231, 190, 232, 482, 23, 105,
    484, 395, 427, 301, 474, 376, 405, 494, 471, 391, 313, 220, 0, 473, 145,
    371, 213, 226, 381, 133, 281, 41, 64, 416, 21, 443, 161, 279, 285, 166,
    124, 116, 449, 26, 165, 168, 193, 57, 208, 181, 89, 146, 182, 126, 125,
    297, 1, 115, 28, 113, 225, 361, 351, 465, 172, 377, 162, 48, 170, 466,
    505, 227, 36, 252, 502, 492, 119, 151, 385, 306, 120, 372, 390, 224, 122,
    270, 100, 418, 433, 329, 365, 396, 91, 222, 55, 496, 498, 103, 51, 293,
    215, 384, 127, 98, 483, 506, 282, 107, 27, 322, 74, 136, 229, 319, 328,
    430, 343, 204, 221, 296, 12, 134, 454, 477, 408, 109, 84, 428, 317, 358,
    394, 299, 205, 171, 288, 143, 68, 267, 216, 435, 149, 485, 434, 141, 464,
    334, 404, 104, 352, 95, 387, 316, 214, 290, 46, 310, 348, 401, 260, 478,
    292, 359, 326, 347, 456, 399, 373, 412, 360, 413], dtype=np.int64)

_KIDX = np.concatenate([_IDX[_B:], _IDX[:_B]]).astype(np.int32)

_INV = np.argsort(_IDX)

_mesh = plsc.VectorSubcoreMesh(core_axis_name="c", subcore_axis_name="s")

_img_out = [
    jax.ShapeDtypeStruct((_Q, _C, _H, _W), jnp.float32),
    jax.ShapeDtypeStruct((_B, _C, _H, _W), jnp.float32),
]

_ROWS_W = _Q // _NW
_HEAD_W = _B // _NW


@functools.partial(pl.kernel, out_type=_img_out, mesh=_mesh,
                   scratch_types=[
                       pltpu.VMEM_SHARED((16, 2, _C, _H, _W), jnp.float32),
                       pltpu.SemaphoreType.DMA,
                       pltpu.SemaphoreType.DMA])
def _sc_img_stream(tbl, batch, newt, deqt, slots, lsem, wsem):
    w = lax.axis_index("s") * 2 + lax.axis_index("c")

    def worker_prog(wi):
        sid = wi // 2
        tasks = []
        for s in range(wi * _ROWS_W, (wi + 1) * _ROWS_W):
            j = int(_INV[s])
            if j < _B:
                tasks.append((tbl, s, deqt, j))
            else:
                tasks.append((tbl, s, newt, j))
        for r in range(wi * _HEAD_W, (wi + 1) * _HEAD_W):
            tasks.append((batch, r, newt, r))

        n = len(tasks)
        hl = [None] * n
        hw = [None] * n
        for i in range(n + 1):
            if i < n:
                if i >= 2:
                    hw[i - 2].wait()
                src_ref, s, _, _ = tasks[i]
                hl[i] = pltpu.async_copy(src_ref.at[s],
                                         slots.at[sid, i % 2], lsem)
            if i >= 1:
                _, _, dst_ref, j = tasks[i - 1]
                hl[i - 1].wait()
                hw[i - 1] = pltpu.async_copy(slots.at[sid, (i - 1) % 2],
                                             dst_ref.at[j], wsem)
        hw[n - 2].wait()
        hw[n - 1].wait()

    for wi in range(_NW):
        @pl.when(w == wi)
        def _(wi=wi):
            worker_prog(wi)


@functools.partial(
    pl.kernel,
    out_type=[
        jax.ShapeDtypeStruct((_Q, _KDP), jnp.float32),
        jax.ShapeDtypeStruct((_B, _KDP), jnp.float32),
    ],
    mesh=_mesh,
    scratch_types=[
        pltpu.VMEM((16,), jnp.int32),
        pltpu.VMEM((16, _KDP), jnp.float32),
        pltpu.SemaphoreType.DMA,
    ],
)
def _sc_ker_stream(kidx, ker2, lr2, newker2, deqker2, kidx_v, kbuf, hsem):
    w = lax.axis_index("s") * 2 + lax.axis_index("c")

    h0 = pltpu.async_copy(lr2.at[pl.ds(w * 2, 2)],
                          newker2.at[pl.ds(w * 2, 2)], hsem)

    def ker_chunk(idx_off, dst, dst_off):
        pltpu.sync_copy(kidx.at[pl.ds(idx_off, 16)], kidx_v)
        pltpu.async_copy(ker2.at[kidx_v], kbuf, hsem).wait()
        pltpu.sync_copy(kbuf, dst.at[pl.ds(dst_off, 16)])

    @pl.when(w < _KA_W)
    def _():
        ker_chunk(w * 16, newker2, _B + w * 16)

    @pl.when(w >= _KA_W)
    def _():
        ker_chunk((_Q - _B) + (w - _KA_W) * 16, deqker2, (w - _KA_W) * 16)

    h0.wait()


def kernel(query, key_img, lr_gt_kernel, queue_q, queue_k, queue_ker):
    ker2 = jnp.pad(queue_ker.reshape(_Q, _KD), ((0, 0), (0, _KDP - _KD)))
    lr2 = jnp.pad(lr_gt_kernel.reshape(_B, _KD), ((0, 0), (0, _KDP - _KD)))
    kidx = jnp.asarray(_KIDX)

    new_qq, q_deq = _sc_img_stream(queue_q, query)
    new_qk, k_deq = _sc_img_stream(queue_k, key_img)
    newker2, deqker2 = _sc_ker_stream(kidx, ker2, lr2)

    new_qker = newker2[:, :_KD].reshape(_Q, 1, _K, _K)
    ker_deq = deqker2[:, :_KD].reshape(_B, 1, _K, _K)
    return (q_deq, k_deq, ker_deq, new_qq, new_qk, new_qker)
